# baseline (speedup 1.0000x reference)
import functools

import jax
import jax.numpy as jnp
from jax import lax
from jax.experimental import pallas as pl
from jax.experimental.pallas import tpu as pltpu
from jax.experimental.pallas import tpu_sc as plsc

_DN = (((1,), (1,)), ((), ()))


def _mm(a, b):
    return lax.dot_general(a, b, _DN, preferred_element_type=jnp.float32)


def _swish(v):
    return v * jax.nn.sigmoid(v)


def _pre_body(x_ref, rbf_ref, Wji, bji, Wkj, bkj, Wr1, Wr2, Wd,
              xji_ref, xkjd_ref):
    x = x_ref[...]
    xji_ref[...] = _swish(_mm(x, Wji[...]) + bji[...])
    rbf_e = _mm(_mm(rbf_ref[...], Wr1[...]), Wr2[...])
    xkj = _swish(_mm(x, Wkj[...]) + bkj[...]) * rbf_e
    d = _swish(_mm(xkj, Wd[...]))
    xkjd_ref[...] = jnp.concatenate([d, jnp.zeros_like(d)], axis=1)


def _sbf_body(sbf_ref, Ws1, Ws2, out_ref):
    e = _mm(_mm(sbf_ref[...], Ws1[...]), Ws2[...])
    out_ref[...] = jnp.concatenate([e, jnp.zeros_like(e)], axis=1)


def _post_body(x_ref, xji_ref, seg_ref, Wup,
               Wb11, bb11, Wb12, bb12, Wl, bl,
               Wa11, ba11, Wa12, ba12, Wa21, ba21, Wa22, ba22,
               out_ref):
    seg = seg_ref[...][:, : Wup.shape[1]]
    h = xji_ref[...] + _swish(_mm(seg, Wup[...]))
    h = h + _swish(_mm(_swish(_mm(h, Wb11[...]) + bb11[...]), Wb12[...]) + bb12[...])
    h = _swish(_mm(h, Wl[...]) + bl[...]) + x_ref[...]
    h = h + _swish(_mm(_swish(_mm(h, Wa11[...]) + ba11[...]), Wa12[...]) + ba12[...])
    h = h + _swish(_mm(_swish(_mm(h, Wa21[...]) + ba21[...]), Wa22[...]) + ba22[...])
    out_ref[...] = h


_NC = 2
_NS = 16
_CRMAX = 6400
_WIN = 1024
_SW = 4
_LL = _SW * _WIN + 160
_INTERPRET = False


def _sc_body(T, E, CR, xkjd, sbfe, ikj, iji, out,
             iji_sw, t_l, kj_vA, ji_vA, kj_vB, ji_vB, dst_gA, dst_gB,
             g_vA, s_vA, g_vB, s_vB, z_v, acc,
             esemA, esemB, gsemA, gsemB, wsem):
    c = lax.axis_index("c")
    s = lax.axis_index("s")
    nw = T // _WIN
    wpp = nw // _NS
    rem = nw % _NS
    my_w = wpp + jnp.where(s < rem, 1, 0)
    sws = (my_w + _SW - 1) // _SW
    half = E // _NC
    nch = half // CR
    stripe = CR // _NS
    zr = z_v.shape[0]
    lanes = lax.broadcasted_iota(jnp.int32, (16,), 0)

    zv = jnp.zeros((16,), jnp.float32)

    def _zb(i, carry):
        z_v[i // 8, pl.ds((i % 8) * 16, 16)] = zv
        return carry

    lax.fori_loop(0, zr * 8, _zb, 0)

    def _chunk(ch, carry_top):
        lo = c * half + ch * CR
        dummy = CR + s
        for i in range(stripe // zr):
            pltpu.sync_copy(z_v, acc.at[pl.ds(s * stripe + i * zr, zr), :])
        plsc.subcore_barrier()

        def _sw(sw, carry0):
            nwi = jnp.minimum(_SW, my_w - sw * _SW)

            def _fire_w(wi, carry):
                widx = (sw * _SW + wi) * _NS + s
                pltpu.async_copy(iji.at[pl.ds(widx * _WIN, _WIN)],
                                 iji_sw.at[pl.ds(wi * _WIN, _WIN)], wsem)
                return carry
            lax.fori_loop(0, nwi, _fire_w, 0)

            def _drain_w(wi, carry):
                pltpu.make_async_copy(iji.at[pl.ds(0, _WIN)],
                                      iji_sw.at[pl.ds(0, _WIN)], wsem).wait()
                return carry
            lax.fori_loop(0, nwi, _drain_w, 0)

            def _scan_grp(g, cnt2):
                sl = pl.ds(g * 16, 16)
                d = iji_sw[sl] - lo
                ok = (d >= 0) & (d < CR)
                wi = g // 64
                widx = (sw * _SW + wi) * _NS + s
                t0 = widx * _WIN + (g - wi * 64) * 16
                plsc.store_compressed(t_l.at[pl.ds(cnt2, 16)],
                                      t0 + lanes, mask=ok)
                return cnt2 + plsc.all_reduce_population_count(ok)[0]

            cnt = lax.fori_loop(0, nwi * (_WIN // 16), _scan_grp, 0)
            spread = (c * _NS + s) * 128 + lanes
            for i in range(8):
                t_l[pl.ds(cnt + i * 16, 16)] = spread + i * 16
            ncp = ((cnt + 63) // 64) * 64

            def _half(kb, kj_v, ji_v, s_b, g_b, dst_b, esem, gsm):
                trow = t_l.at[pl.ds(kb, 64)]
                pltpu.async_copy(ikj.at[trow], kj_v, esem)
                pltpu.async_copy(iji.at[trow], ji_v, esem)
                pltpu.async_copy(sbfe.at[trow], s_b, esem)

                def fire_row_gather():
                    pltpu.make_async_copy(ikj.at[trow], kj_v, esem).wait()
                    pltpu.make_async_copy(iji.at[trow], ji_v, esem).wait()
                    pltpu.make_async_copy(sbfe.at[trow], s_b, esem).wait()
                    return pltpu.async_copy(xkjd.at[kj_v], g_b, gsm)

                def finish(cg):
                    npad = cnt - kb
                    for j in range(4):
                        sl = pl.ds(j * 16, 16)
                        d = ji_v[sl] - lo
                        ok = (d >= 0) & (d < CR) & (j * 16 + lanes < npad)
                        dst_b[0, sl] = jnp.where(ok, d, dummy)
                    cg.wait()

                    def _mul(r, carry2):
                        for j in range(4):
                            sl = pl.ds(j * 16, 16)
                            s_b[r, sl] = g_b[r, sl] * s_b[r, sl]
                        return carry2
                    lax.fori_loop(0, 64, _mul, 0)
                    pltpu.sync_copy(s_b, acc.at[dst_b.at[0]], add=True)

                return fire_row_gather, finish

            def _burst(p, carry1):
                fA, finA = _half(p * 128, kj_vA, ji_vA, s_vA, g_vA,
                                 dst_gA, esemA, gsemA)
                fB, finB = _half(p * 128 + 64, kj_vB, ji_vB, s_vB, g_vB,
                                 dst_gB, esemB, gsemB)
                cgA = fA()
                cgB = fB()
                finA(cgA)
                finB(cgB)
                return carry1

            lax.fori_loop(0, (ncp + 127) // 128, _burst, 0)
            return carry0

        lax.fori_loop(0, sws, _sw, 0)
        plsc.subcore_barrier()
        pltpu.sync_copy(acc.at[pl.ds(s * stripe, stripe), :],
                        out.at[pl.ds(lo + s * stripe, stripe), :])
        plsc.subcore_barrier()
        return carry_top

    lax.fori_loop(0, nch, _chunk, 0)


def _segment_gather_scatter(xkjd, sbfe, idx_kj, idx_ji, E, T, HID):
    half = E // _NC
    CR = min(_CRMAX, half)
    assert half % CR == 0 and (CR // _NS) % 8 == 0
    zr = max(d for d in range(8, 129, 8) if (CR // _NS) % d == 0)
    mesh = plsc.VectorSubcoreMesh(core_axis_name="c", subcore_axis_name="s",
                                  num_cores=_NC, num_subcores=_NS)
    fn = pl.kernel(
        functools.partial(_sc_body, T, E, CR),
        out_type=jax.ShapeDtypeStruct((E, HID), jnp.float32),
        mesh=mesh,
        interpret=_INTERPRET,
        compiler_params=pltpu.CompilerParams(needs_layout_passes=False),
        scratch_types=[
            pltpu.VMEM((_SW * _WIN,), jnp.int32),
            pltpu.VMEM((_LL,), jnp.int32),
            pltpu.VMEM((64,), jnp.int32),
            pltpu.VMEM((64,), jnp.int32),
            pltpu.VMEM((64,), jnp.int32),
            pltpu.VMEM((64,), jnp.int32),
            pltpu.VMEM((1, 64), jnp.int32),
            pltpu.VMEM((1, 64), jnp.int32),
            pltpu.VMEM((64, HID), jnp.float32),
            pltpu.VMEM((64, HID), jnp.float32),
            pltpu.VMEM((64, HID), jnp.float32),
            pltpu.VMEM((64, HID), jnp.float32),
            pltpu.VMEM((zr, HID), jnp.float32),
            pltpu.VMEM_SHARED((CR + _NS, HID), jnp.float32),
            pltpu.SemaphoreType.DMA,
            pltpu.SemaphoreType.DMA,
            pltpu.SemaphoreType.DMA,
            pltpu.SemaphoreType.DMA,
            pltpu.SemaphoreType.DMA,
        ],
    )
    return fn(xkjd, sbfe, idx_kj, idx_ji)


def kernel(x, rbf, sbf, idx_kj, idx_ji,
           W_rbf1, W_rbf2, W_sbf1, W_sbf2,
           W_kj, b_kj, W_ji, b_ji, W_down, W_up,
           Wb1_1, bb1_1, Wb1_2, bb1_2,
           W_lin, b_lin,
           Wa1_1, ba1_1, Wa1_2, ba1_2,
           Wa2_1, ba2_1, Wa2_2, ba2_2):
    E, HID = x.shape
    T = idx_kj.shape[0]
    INTD = W_down.shape[0]
    NR = rbf.shape[1]
    NSR = sbf.shape[1]
    BE = 2000
    BT = 2048

    b_kj2 = b_kj.reshape(1, HID)
    b_ji2 = b_ji.reshape(1, HID)

    full = lambda shape: pl.BlockSpec(shape, lambda i: (0, 0))

    xji, xkjd = pl.pallas_call(
        _pre_body,
        grid=(E // BE,),
        in_specs=[
            pl.BlockSpec((BE, HID), lambda i: (i, 0)),
            pl.BlockSpec((BE, NR), lambda i: (i, 0)),
            full((HID, HID)), full((1, HID)),
            full((HID, HID)), full((1, HID)),
            full(W_rbf1.shape), full(W_rbf2.shape), full(W_down.shape),
        ],
        out_specs=[
            pl.BlockSpec((BE, HID), lambda i: (i, 0)),
            pl.BlockSpec((BE, HID), lambda i: (i, 0)),
        ],
        out_shape=[
            jax.ShapeDtypeStruct((E, HID), jnp.float32),
            jax.ShapeDtypeStruct((E, HID), jnp.float32),
        ],
        interpret=_INTERPRET,
    )(x, rbf, W_ji, b_ji2, W_kj, b_kj2, W_rbf1, W_rbf2, W_down)

    sbfe = pl.pallas_call(
        _sbf_body,
        grid=(T // BT,),
        in_specs=[
            pl.BlockSpec((BT, NSR), lambda i: (i, 0)),
            full(W_sbf1.shape), full(W_sbf2.shape),
        ],
        out_specs=pl.BlockSpec((BT, HID), lambda i: (i, 0)),
        out_shape=jax.ShapeDtypeStruct((T, HID), jnp.float32),
        interpret=_INTERPRET,
    )(sbf, W_sbf1, W_sbf2)

    seg = _segment_gather_scatter(xkjd, sbfe, idx_kj, idx_ji, E, T, HID)

    biases = dict(
        bb11=bb1_1.reshape(1, HID), bb12=bb1_2.reshape(1, HID),
        bl=b_lin.reshape(1, HID),
        ba11=ba1_1.reshape(1, HID), ba12=ba1_2.reshape(1, HID),
        ba21=ba2_1.reshape(1, HID), ba22=ba2_2.reshape(1, HID),
    )

    out = pl.pallas_call(
        _post_body,
        grid=(E // BE,),
        in_specs=[
            pl.BlockSpec((BE, HID), lambda i: (i, 0)),
            pl.BlockSpec((BE, HID), lambda i: (i, 0)),
            pl.BlockSpec((BE, HID), lambda i: (i, 0)),
            full(W_up.shape),
            full((HID, HID)), full((1, HID)),
            full((HID, HID)), full((1, HID)),
            full((HID, HID)), full((1, HID)),
            full((HID, HID)), full((1, HID)),
            full((HID, HID)), full((1, HID)),
            full((HID, HID)), full((1, HID)),
            full((HID, HID)), full((1, HID)),
        ],
        out_specs=pl.BlockSpec((BE, HID), lambda i: (i, 0)),
        out_shape=jax.ShapeDtypeStruct((E, HID), jnp.float32),
        interpret=_INTERPRET,
    )(x, xji, seg, W_up,
      Wb1_1, biases["bb11"], Wb1_2, biases["bb12"], W_lin, biases["bl"],
      Wa1_1, biases["ba11"], Wa1_2, biases["ba12"],
      Wa2_1, biases["ba21"], Wa2_2, biases["ba22"])

    return out


# speedup vs baseline: 1.0408x; 1.0408x over previous
import functools

import jax
import jax.numpy as jnp
from jax import lax
from jax.experimental import pallas as pl
from jax.experimental.pallas import tpu as pltpu
from jax.experimental.pallas import tpu_sc as plsc

_DN = (((1,), (1,)), ((), ()))


def _mm(a, b):
    return lax.dot_general(a, b, _DN, preferred_element_type=jnp.float32)


def _swish(v):
    return v * jax.nn.sigmoid(v)


def _pre_body(x_ref, rbf_ref, Wji, bji, Wkj, bkj, Wr1, Wr2, Wd,
              xji_ref, xkjd_ref):
    x = x_ref[...]
    xji_ref[...] = _swish(_mm(x, Wji[...]) + bji[...])
    rbf_e = _mm(_mm(rbf_ref[...], Wr1[...]), Wr2[...])
    xkj = _swish(_mm(x, Wkj[...]) + bkj[...]) * rbf_e
    d = _swish(_mm(xkj, Wd[...]))
    xkjd_ref[...] = jnp.concatenate([d, jnp.zeros_like(d)], axis=1)


def _sbf_body(sbf_ref, Ws1, Ws2, out_ref):
    e = _mm(_mm(sbf_ref[...], Ws1[...]), Ws2[...])
    out_ref[...] = jnp.concatenate([e, jnp.zeros_like(e)], axis=1)


def _post_body(x_ref, xji_ref, seg_ref, Wup,
               Wb11, bb11, Wb12, bb12, Wl, bl,
               Wa11, ba11, Wa12, ba12, Wa21, ba21, Wa22, ba22,
               out_ref):
    seg = seg_ref[...][:, : Wup.shape[1]]
    h = xji_ref[...] + _swish(_mm(seg, Wup[...]))
    h = h + _swish(_mm(_swish(_mm(h, Wb11[...]) + bb11[...]), Wb12[...]) + bb12[...])
    h = _swish(_mm(h, Wl[...]) + bl[...]) + x_ref[...]
    h = h + _swish(_mm(_swish(_mm(h, Wa11[...]) + ba11[...]), Wa12[...]) + ba12[...])
    h = h + _swish(_mm(_swish(_mm(h, Wa21[...]) + ba21[...]), Wa22[...]) + ba22[...])
    out_ref[...] = h


_NC = 2
_NS = 16
_CRMAX = 6400
_WIN = 1024
_SW = 8
_LL = _SW * _WIN + 160
_INTERPRET = False


def _sc_body(T, E, CR, xkjd, sbfe, ikj, iji, out,
             iji_sw, t_l, kj_vals, ji_vals, dst_g2, g_v, s_v, z_v, acc,
             gsem, wsem):
    c = lax.axis_index("c")
    s = lax.axis_index("s")
    nw = T // _WIN
    wpp = nw // _NS
    rem = nw % _NS
    my_w = wpp + jnp.where(s < rem, 1, 0)
    sws = (my_w + _SW - 1) // _SW
    half = E // _NC
    nch = half // CR
    stripe = CR // _NS
    zr = z_v.shape[0]
    lanes = lax.broadcasted_iota(jnp.int32, (16,), 0)

    zv = jnp.zeros((16,), jnp.float32)

    def _zb(i, carry):
        z_v[i // 8, pl.ds((i % 8) * 16, 16)] = zv
        return carry

    lax.fori_loop(0, zr * 8, _zb, 0)

    def _chunk(ch, carry_top):
        lo = c * half + ch * CR
        dummy = CR + s
        for i in range(stripe // zr):
            pltpu.sync_copy(z_v, acc.at[pl.ds(s * stripe + i * zr, zr), :])
        plsc.subcore_barrier()

        def _sw(sw, carry0):
            nwi = jnp.minimum(_SW, my_w - sw * _SW)

            def _fire_w(wi, carry):
                widx = (sw * _SW + wi) * _NS + s
                pltpu.async_copy(iji.at[pl.ds(widx * _WIN, _WIN)],
                                 iji_sw.at[pl.ds(wi * _WIN, _WIN)], wsem)
                return carry
            lax.fori_loop(0, nwi, _fire_w, 0)

            def _drain_w(wi, carry):
                pltpu.make_async_copy(iji.at[pl.ds(0, _WIN)],
                                      iji_sw.at[pl.ds(0, _WIN)], wsem).wait()
                return carry
            lax.fori_loop(0, nwi, _drain_w, 0)

            def _scan_grp(g, cnt2):
                sl = pl.ds(g * 16, 16)
                d = iji_sw[sl] - lo
                ok = (d >= 0) & (d < CR)
                wi = g // 64
                widx = (sw * _SW + wi) * _NS + s
                t0 = widx * _WIN + (g - wi * 64) * 16
                plsc.store_compressed(t_l.at[pl.ds(cnt2, 16)],
                                      t0 + lanes, mask=ok)
                return cnt2 + plsc.all_reduce_population_count(ok)[0]

            cnt = lax.fori_loop(0, nwi * (_WIN // 16), _scan_grp, 0)
            spread = (c * _NS + s) * 128 + lanes
            for i in range(8):
                t_l[pl.ds(cnt + i * 16, 16)] = spread + i * 16
            ncp = ((cnt + 127) // 128) * 128

            def _burst(b, carry1):
                kb = b * 128
                trow = t_l.at[pl.ds(kb, 128)]
                ck = pltpu.async_copy(ikj.at[trow], kj_vals, gsem)
                cj = pltpu.async_copy(iji.at[trow], ji_vals, gsem)
                cs = pltpu.async_copy(sbfe.at[trow], s_v, gsem)
                ck.wait()
                cg = pltpu.async_copy(xkjd.at[kj_vals], g_v, gsem)
                cj.wait()
                npad = cnt - kb
                for j in range(8):
                    sl = pl.ds(j * 16, 16)
                    d = ji_vals[sl] - lo
                    ok = (d >= 0) & (d < CR) & (j * 16 + lanes < npad)
                    dst_g2[0, sl] = jnp.where(ok, d, dummy)
                cs.wait()
                cg.wait()

                def _mul(r, carry2):
                    for u in range(4):
                        for j in range(4):
                            sl = pl.ds(j * 16, 16)
                            s_v[r * 4 + u, sl] = g_v[r * 4 + u, sl] * s_v[r * 4 + u, sl]
                    return carry2
                lax.fori_loop(0, 32, _mul, 0)

                pltpu.sync_copy(s_v, acc.at[dst_g2.at[0]], add=True)
                return carry1

            lax.fori_loop(0, ncp // 128, _burst, 0)
            return carry0

        lax.fori_loop(0, sws, _sw, 0)
        plsc.subcore_barrier()
        pltpu.sync_copy(acc.at[pl.ds(s * stripe, stripe), :],
                        out.at[pl.ds(lo + s * stripe, stripe), :])
        plsc.subcore_barrier()
        return carry_top

    lax.fori_loop(0, nch, _chunk, 0)


def _segment_gather_scatter(xkjd, sbfe, idx_kj, idx_ji, E, T, HID):
    half = E // _NC
    CR = min(_CRMAX, half)
    assert half % CR == 0 and (CR // _NS) % 8 == 0
    zr = max(d for d in range(8, 129, 8) if (CR // _NS) % d == 0)
    mesh = plsc.VectorSubcoreMesh(core_axis_name="c", subcore_axis_name="s",
                                  num_cores=_NC, num_subcores=_NS)
    fn = pl.kernel(
        functools.partial(_sc_body, T, E, CR),
        out_type=jax.ShapeDtypeStruct((E, HID), jnp.float32),
        mesh=mesh,
        interpret=_INTERPRET,
        compiler_params=pltpu.CompilerParams(needs_layout_passes=False),
        scratch_types=[
            pltpu.VMEM((_SW * _WIN,), jnp.int32),
            pltpu.VMEM((_LL,), jnp.int32),
            pltpu.VMEM((128,), jnp.int32),
            pltpu.VMEM((128,), jnp.int32),
            pltpu.VMEM((1, 128), jnp.int32),
            pltpu.VMEM((128, HID), jnp.float32),
            pltpu.VMEM((128, HID), jnp.float32),
            pltpu.VMEM((zr, HID), jnp.float32),
            pltpu.VMEM_SHARED((CR + _NS, HID), jnp.float32),
            pltpu.SemaphoreType.DMA,
            pltpu.SemaphoreType.DMA,
        ],
    )
    return fn(xkjd, sbfe, idx_kj, idx_ji)


def kernel(x, rbf, sbf, idx_kj, idx_ji,
           W_rbf1, W_rbf2, W_sbf1, W_sbf2,
           W_kj, b_kj, W_ji, b_ji, W_down, W_up,
           Wb1_1, bb1_1, Wb1_2, bb1_2,
           W_lin, b_lin,
           Wa1_1, ba1_1, Wa1_2, ba1_2,
           Wa2_1, ba2_1, Wa2_2, ba2_2):
    E, HID = x.shape
    T = idx_kj.shape[0]
    INTD = W_down.shape[0]
    NR = rbf.shape[1]
    NSR = sbf.shape[1]
    BE = 2000
    BT = 2048

    b_kj2 = b_kj.reshape(1, HID)
    b_ji2 = b_ji.reshape(1, HID)

    full = lambda shape: pl.BlockSpec(shape, lambda i: (0, 0))

    xji, xkjd = pl.pallas_call(
        _pre_body,
        grid=(E // BE,),
        in_specs=[
            pl.BlockSpec((BE, HID), lambda i: (i, 0)),
            pl.BlockSpec((BE, NR), lambda i: (i, 0)),
            full((HID, HID)), full((1, HID)),
            full((HID, HID)), full((1, HID)),
            full(W_rbf1.shape), full(W_rbf2.shape), full(W_down.shape),
        ],
        out_specs=[
            pl.BlockSpec((BE, HID), lambda i: (i, 0)),
            pl.BlockSpec((BE, HID), lambda i: (i, 0)),
        ],
        out_shape=[
            jax.ShapeDtypeStruct((E, HID), jnp.float32),
            jax.ShapeDtypeStruct((E, HID), jnp.float32),
        ],
        interpret=_INTERPRET,
    )(x, rbf, W_ji, b_ji2, W_kj, b_kj2, W_rbf1, W_rbf2, W_down)

    sbfe = pl.pallas_call(
        _sbf_body,
        grid=(T // BT,),
        in_specs=[
            pl.BlockSpec((BT, NSR), lambda i: (i, 0)),
            full(W_sbf1.shape), full(W_sbf2.shape),
        ],
        out_specs=pl.BlockSpec((BT, HID), lambda i: (i, 0)),
        out_shape=jax.ShapeDtypeStruct((T, HID), jnp.float32),
        interpret=_INTERPRET,
    )(sbf, W_sbf1, W_sbf2)

    seg = _segment_gather_scatter(xkjd, sbfe, idx_kj, idx_ji, E, T, HID)

    biases = dict(
        bb11=bb1_1.reshape(1, HID), bb12=bb1_2.reshape(1, HID),
        bl=b_lin.reshape(1, HID),
        ba11=ba1_1.reshape(1, HID), ba12=ba1_2.reshape(1, HID),
        ba21=ba2_1.reshape(1, HID), ba22=ba2_2.reshape(1, HID),
    )

    out = pl.pallas_call(
        _post_body,
        grid=(E // BE,),
        in_specs=[
            pl.BlockSpec((BE, HID), lambda i: (i, 0)),
            pl.BlockSpec((BE, HID), lambda i: (i, 0)),
            pl.BlockSpec((BE, HID), lambda i: (i, 0)),
            full(W_up.shape),
            full((HID, HID)), full((1, HID)),
            full((HID, HID)), full((1, HID)),
            full((HID, HID)), full((1, HID)),
            full((HID, HID)), full((1, HID)),
            full((HID, HID)), full((1, HID)),
            full((HID, HID)), full((1, HID)),
            full((HID, HID)), full((1, HID)),
        ],
        out_specs=pl.BlockSpec((BE, HID), lambda i: (i, 0)),
        out_shape=jax.ShapeDtypeStruct((E, HID), jnp.float32),
        interpret=_INTERPRET,
    )(x, xji, seg, W_up,
      Wb1_1, biases["bb11"], Wb1_2, biases["bb12"], W_lin, biases["bl"],
      Wa1_1, biases["ba11"], Wa1_2, biases["ba12"],
      Wa2_1, biases["ba21"], Wa2_2, biases["ba22"])

    return out
